# baseline (speedup 1.0000x reference)
import jax
import jax.numpy as jnp
from jax import lax
from jax.experimental import pallas as pl
from jax.experimental.pallas import tpu as pltpu

B, T, DIM = 2, 4096, 1024
H = 16
DH = DIM // H
BSZ = 64
NB = T // BSZ
BH = B * H
TB = 512
SCALE = DH ** -0.5
BF = jnp.bfloat16
F32 = jnp.float32


def _qkv_kernel(x_ref, w_ref, q_ref, k_ref, v_ref):
    out = jnp.dot(x_ref[0], w_ref[...], preferred_element_type=F32)
    out = out.astype(BF)
    for h in range(H):
        q_ref[h, :, :] = out[:, h * DH:(h + 1) * DH]
        k_ref[h, :, :] = out[:, DIM + h * DH:DIM + (h + 1) * DH]
        v_ref[h, :, :] = out[:, 2 * DIM + h * DH:2 * DIM + (h + 1) * DH]


def _attn_kernel(q_ref, k_ref, v_ref, kf_ref, vf_ref, o_ref):
    outs = []
    for p in range(2):
        q3 = q_ref[p].reshape(NB, BSZ, DH)
        k3 = k_ref[p].reshape(NB, BSZ, DH)
        v3 = v_ref[p].reshape(NB, BSZ, DH)
        kf = kf_ref[p]
        vf = vf_ref[p]
        sq = jnp.sum(q3.astype(F32), axis=1) * (1.0 / BSZ)
        sk = jnp.sum(k3.astype(F32), axis=1) * (1.0 / BSZ)
        logits = lax.dot_general(sq, sk, (((1,), (1,)), ((), ())),
                                 preferred_element_type=F32) * SCALE
        lm = jnp.max(logits, axis=-1, keepdims=True)
        le = jnp.exp(logits - lm)
        R = (le / jnp.sum(le, axis=-1, keepdims=True)).astype(BF)
        kr3 = jnp.dot(R, kf, preferred_element_type=F32
                      ).astype(BF).reshape(NB, BSZ, DH)
        vr3 = jnp.dot(R, vf, preferred_element_type=F32
                      ).astype(BF).reshape(NB, BSZ, DH)
        kcat = jnp.concatenate([kr3, k3], axis=1)
        vcat = jnp.concatenate([vr3, v3], axis=1)
        d = lax.dot_general(q3, kcat, (((2,), (2,)), ((0,), (0,))),
                            preferred_element_type=F32) * SCALE
        m = jnp.max(d, axis=-1, keepdims=True)
        e = jnp.exp(d - m)
        o = lax.dot_general(e.astype(BF), vcat, (((2,), (1,)), ((0,), (0,))),
                            preferred_element_type=F32)
        o = o / jnp.sum(e, axis=-1, keepdims=True)
        outs.append(o.reshape(T, DH))
    o_ref[0] = jnp.concatenate(outs, axis=1).astype(BF)


def _out_kernel(x_ref, w_ref, b_ref, o_ref):
    o_ref[...] = (jnp.dot(x_ref[...], w_ref[...],
                          preferred_element_type=F32) + b_ref[...])


def kernel(x, Wq, Wkv, Wout, bout):
    W3 = jnp.concatenate([Wq, Wkv], axis=1).astype(BF)
    x16 = x.astype(BF)

    q, k, v = pl.pallas_call(
        _qkv_kernel,
        grid=(B, T // TB),
        in_specs=[
            pl.BlockSpec((1, TB, DIM), lambda b, t: (b, t, 0)),
            pl.BlockSpec((DIM, 3 * DIM), lambda b, t: (0, 0)),
        ],
        out_specs=[
            pl.BlockSpec((H, TB, DH), lambda b, t: (b, t, 0)),
            pl.BlockSpec((H, TB, DH), lambda b, t: (b, t, 0)),
            pl.BlockSpec((H, TB, DH), lambda b, t: (b, t, 0)),
        ],
        out_shape=[jax.ShapeDtypeStruct((BH, T, DH), BF)] * 3,
    )(x16, W3)

    kf = k.reshape(BH, NB, BSZ * DH)
    vf = v.reshape(BH, NB, BSZ * DH)

    attn = pl.pallas_call(
        _attn_kernel,
        grid=(BH // 2,),
        in_specs=[pl.BlockSpec((2, T, DH), lambda i: (i, 0, 0))] * 3
        + [pl.BlockSpec((2, NB, BSZ * DH), lambda i: (i, 0, 0))] * 2,
        out_specs=pl.BlockSpec((1, T, 2 * DH),
                               lambda i: (i // (H // 2), 0, i % (H // 2))),
        out_shape=jax.ShapeDtypeStruct((B, T, DIM), BF),
    )(q, k, v, kf, vf)

    out = pl.pallas_call(
        _out_kernel,
        grid=(B * T // TB,),
        in_specs=[
            pl.BlockSpec((TB, DIM), lambda i: (i, 0)),
            pl.BlockSpec((DIM, DIM), lambda i: (0, 0)),
            pl.BlockSpec((1, DIM), lambda i: (0, 0)),
        ],
        out_specs=pl.BlockSpec((TB, DIM), lambda i: (i, 0)),
        out_shape=jax.ShapeDtypeStruct((B * T, DIM), F32),
    )(attn.reshape(B * T, DIM), Wout.astype(BF), bout.reshape(1, DIM))

    return out.reshape(B, T, DIM)


# speedup vs baseline: 1.2330x; 1.2330x over previous
import jax
import jax.numpy as jnp
from jax import lax
from jax.experimental import pallas as pl
from jax.experimental.pallas import tpu as pltpu

B, T, DIM = 2, 4096, 1024
H = 16
DH = DIM // H
BSZ = 64
NB = T // BSZ
BH = B * H
TB = 512
SCALE = DH ** -0.5
BF = jnp.bfloat16
F32 = jnp.float32


def _qkv_kernel(x_ref, w_ref, q_ref, k_ref, v_ref):
    out = jnp.dot(x_ref[0], w_ref[...], preferred_element_type=F32)
    out = out.astype(BF)
    for h in range(H):
        q_ref[h, :, :] = out[:, h * DH:(h + 1) * DH]
        k_ref[h, :, :] = out[:, DIM + h * DH:DIM + (h + 1) * DH]
        v_ref[h, :, :] = out[:, 2 * DIM + h * DH:2 * DIM + (h + 1) * DH]


def _attn_kernel(q_ref, k_ref, v_ref, o_ref):
    outs = []
    for p in range(2):
        q3 = q_ref[p].reshape(NB, BSZ, DH)
        k3 = k_ref[p].reshape(NB, BSZ, DH)
        v3 = v_ref[p].reshape(NB, BSZ, DH)
        sq = jnp.sum(q3.astype(F32), axis=1) * (1.0 / BSZ)
        sk = jnp.sum(k3.astype(F32), axis=1) * (1.0 / BSZ)
        logits = lax.dot_general(sq, sk, (((1,), (1,)), ((), ())),
                                 preferred_element_type=F32) * SCALE
        lm = jnp.max(logits, axis=-1, keepdims=True)
        le = jnp.exp(logits - lm)
        R = (le / jnp.sum(le, axis=-1, keepdims=True)).astype(BF)
        kr3 = lax.dot_general(R, k3, (((1,), (0,)), ((), ())),
                              preferred_element_type=F32).astype(BF)
        vr3 = lax.dot_general(R, v3, (((1,), (0,)), ((), ())),
                              preferred_element_type=F32).astype(BF)
        kcat = jnp.concatenate([kr3, k3], axis=1)
        vcat = jnp.concatenate([vr3, v3], axis=1)
        d = lax.dot_general(q3, kcat, (((2,), (2,)), ((0,), (0,))),
                            preferred_element_type=F32) * SCALE
        m = jnp.max(d, axis=-1, keepdims=True)
        e = jnp.exp(d - m)
        o = lax.dot_general(e.astype(BF), vcat, (((2,), (1,)), ((0,), (0,))),
                            preferred_element_type=F32)
        o = o / jnp.sum(e, axis=-1, keepdims=True)
        outs.append(o.reshape(T, DH))
    o_ref[0] = jnp.concatenate(outs, axis=1).astype(BF)


def _out_kernel(x_ref, w_ref, b_ref, o_ref):
    o_ref[...] = (jnp.dot(x_ref[...], w_ref[...],
                          preferred_element_type=F32) + b_ref[...])


def kernel(x, Wq, Wkv, Wout, bout):
    W3 = jnp.concatenate([Wq, Wkv], axis=1).astype(BF)
    x16 = x.astype(BF)

    q, k, v = pl.pallas_call(
        _qkv_kernel,
        grid=(B, T // TB),
        in_specs=[
            pl.BlockSpec((1, TB, DIM), lambda b, t: (b, t, 0)),
            pl.BlockSpec((DIM, 3 * DIM), lambda b, t: (0, 0)),
        ],
        out_specs=[
            pl.BlockSpec((H, TB, DH), lambda b, t: (b, t, 0)),
            pl.BlockSpec((H, TB, DH), lambda b, t: (b, t, 0)),
            pl.BlockSpec((H, TB, DH), lambda b, t: (b, t, 0)),
        ],
        out_shape=[jax.ShapeDtypeStruct((BH, T, DH), BF)] * 3,
        compiler_params=pltpu.CompilerParams(
            dimension_semantics=("parallel", "parallel")),
    )(x16, W3)

    attn = pl.pallas_call(
        _attn_kernel,
        grid=(BH // 2,),
        in_specs=[pl.BlockSpec((2, T, DH), lambda i: (i, 0, 0))] * 3,
        out_specs=pl.BlockSpec((1, T, 2 * DH),
                               lambda i: (i // (H // 2), 0, i % (H // 2))),
        out_shape=jax.ShapeDtypeStruct((B, T, DIM), BF),
        compiler_params=pltpu.CompilerParams(
            dimension_semantics=("parallel",)),
    )(q, k, v)

    out = pl.pallas_call(
        _out_kernel,
        grid=(B * T // TB,),
        in_specs=[
            pl.BlockSpec((TB, DIM), lambda i: (i, 0)),
            pl.BlockSpec((DIM, DIM), lambda i: (0, 0)),
            pl.BlockSpec((1, DIM), lambda i: (0, 0)),
        ],
        out_specs=pl.BlockSpec((TB, DIM), lambda i: (i, 0)),
        out_shape=jax.ShapeDtypeStruct((B * T, DIM), F32),
        compiler_params=pltpu.CompilerParams(
            dimension_semantics=("parallel",)),
    )(attn.reshape(B * T, DIM), Wout.astype(BF), bout.reshape(1, DIM))

    return out.reshape(B, T, DIM)
